# Initial kernel scaffold; baseline (speedup 1.0000x reference)
import jax
import jax.numpy as jnp
from jax.experimental import pallas as pl


def kernel(x, edge_index, W1, b1, W2, b2):
    raise NotImplementedError("write your pallas kernel here")


# speedup vs baseline: 15.9765x; 15.9765x over previous
import functools

import jax
import jax.numpy as jnp
from jax import lax
from jax.experimental import pallas as pl
from jax.experimental.pallas import tpu as pltpu
from jax.experimental.pallas import tpu_sc as plsc

NC = 2
NS = 16
NW = NC * NS
LANES = 16
CHUNK = 128
PAD_ROWS = 16


def _mesh():
    return plsc.VectorSubcoreMesh(core_axis_name="c", subcore_axis_name="s")


def _make_deg_kernel(NP, EPT):
    nz = NP // NS

    @functools.partial(
        pl.kernel,
        out_type=jax.ShapeDtypeStruct((NC * NP,), jnp.float32),
        mesh=_mesh(),
        scratch_types=[
            pltpu.VMEM((CHUNK,), jnp.int32),
            pltpu.VMEM((CHUNK,), jnp.float32),
            pltpu.VMEM((NP // NS,), jnp.float32),
            pltpu.VMEM_SHARED((NP,), jnp.float32),
            pltpu.SemaphoreType.DMA,
        ],
    )
    def deg_kernel(dst_hbm, out_hbm, idx_v, ones_v, row_v, acc_sh, sem):
        cid = lax.axis_index("c")
        sid = lax.axis_index("s")
        wid = sid * NC + cid

        @pl.loop(0, CHUNK // LANES)
        def _(i):
            ones_v[pl.ds(i * LANES, LANES)] = jnp.full((LANES,), 1.0, jnp.float32)

        @pl.loop(0, nz // LANES)
        def _(i):
            row_v[pl.ds(i * LANES, LANES)] = jnp.zeros((LANES,), jnp.float32)

        pltpu.sync_copy(row_v, acc_sh.at[pl.ds(sid * nz, nz)])
        plsc.subcore_barrier()

        base = wid * EPT

        @pl.loop(0, EPT // CHUNK)
        def _(c):
            pltpu.sync_copy(dst_hbm.at[pl.ds(base + c * CHUNK, CHUNK)], idx_v)
            pltpu.sync_copy(ones_v, acc_sh.at[idx_v], add=True)

        plsc.subcore_barrier()
        pltpu.sync_copy(acc_sh.at[pl.ds(sid * nz, nz)], row_v)
        pltpu.sync_copy(row_v, out_hbm.at[pl.ds(cid * NP + sid * nz, nz)])

    return deg_kernel


def _make_scatter_kernel(NP, EPT, D):
    nz = NP // NS

    @functools.partial(
        pl.kernel,
        out_type=jax.ShapeDtypeStruct((NC, NP, D), jnp.float32),
        mesh=_mesh(),
        scratch_types=[
            pltpu.VMEM((CHUNK,), jnp.int32),
            pltpu.VMEM((CHUNK,), jnp.int32),
            pltpu.VMEM((CHUNK, D), jnp.float32),
            pltpu.VMEM_SHARED((NP, D), jnp.float32),
            pltpu.SemaphoreType.DMA,
        ],
    )
    def scat_kernel(h_hbm, src_hbm, dst_hbm, zeros_hbm, out_hbm,
                    si_v, di_v, rows_v, acc_sh, sem):
        cid = lax.axis_index("c")
        sid = lax.axis_index("s")
        wid = sid * NC + cid

        pltpu.sync_copy(zeros_hbm.at[pl.ds(sid * nz, nz)],
                        acc_sh.at[pl.ds(sid * nz, nz)])
        plsc.subcore_barrier()

        base = wid * EPT

        @pl.loop(0, EPT // CHUNK)
        def _(c):
            off = base + c * CHUNK
            pltpu.sync_copy(src_hbm.at[pl.ds(off, CHUNK)], si_v)
            pltpu.sync_copy(dst_hbm.at[pl.ds(off, CHUNK)], di_v)
            pltpu.async_copy(h_hbm.at[si_v], rows_v, sem).wait()
            pltpu.sync_copy(rows_v, acc_sh.at[di_v], add=True)

        plsc.subcore_barrier()
        pltpu.sync_copy(acc_sh.at[pl.ds(sid * nz, nz)],
                        out_hbm.at[cid, pl.ds(sid * nz, nz)])

    return scat_kernel


def _matmul(x, W):
    NP, D = x.shape
    B = NP // 8

    def body(x_ref, w_ref, o_ref):
        o_ref[...] = jnp.dot(x_ref[...], w_ref[...],
                             preferred_element_type=jnp.float32)

    return pl.pallas_call(
        body,
        grid=(8,),
        in_specs=[pl.BlockSpec((B, D), lambda i: (i, 0)),
                  pl.BlockSpec((D, D), lambda i: (0, 0))],
        out_specs=pl.BlockSpec((B, D), lambda i: (i, 0)),
        out_shape=jax.ShapeDtypeStruct((NP, D), jnp.float32),
    )(x, W)


def _prep(degp, g1):
    _, NP, _ = degp.shape
    D = g1.shape[1]
    B = NP // 8

    def body(d_ref, g_ref, di_ref, h_ref):
        d = d_ref[0] + d_ref[1]
        di = lax.rsqrt(d + 1.0)
        di_ref[...] = di
        h_ref[...] = di * g_ref[...]

    return pl.pallas_call(
        body,
        grid=(8,),
        in_specs=[pl.BlockSpec((2, B, 1), lambda i: (0, i, 0)),
                  pl.BlockSpec((B, D), lambda i: (i, 0))],
        out_specs=[pl.BlockSpec((B, 1), lambda i: (i, 0)),
                   pl.BlockSpec((B, D), lambda i: (i, 0))],
        out_shape=[jax.ShapeDtypeStruct((NP, 1), jnp.float32),
                   jax.ShapeDtypeStruct((NP, D), jnp.float32)],
    )(degp, g1)


def _mid(y, ht, dinv, b, W):
    _, NP, D = y.shape
    B = NP // 8

    def body(y_ref, h_ref, di_ref, b_ref, w_ref, o_ref):
        s = y_ref[0] + y_ref[1] + h_ref[...]
        z = jnp.maximum(di_ref[...] * s + b_ref[...], 0.0)
        o_ref[...] = di_ref[...] * jnp.dot(z, w_ref[...],
                                           preferred_element_type=jnp.float32)

    return pl.pallas_call(
        body,
        grid=(8,),
        in_specs=[pl.BlockSpec((2, B, D), lambda i: (0, i, 0)),
                  pl.BlockSpec((B, D), lambda i: (i, 0)),
                  pl.BlockSpec((B, 1), lambda i: (i, 0)),
                  pl.BlockSpec((1, D), lambda i: (0, 0)),
                  pl.BlockSpec((D, D), lambda i: (0, 0))],
        out_specs=pl.BlockSpec((B, D), lambda i: (i, 0)),
        out_shape=jax.ShapeDtypeStruct((NP, D), jnp.float32),
    )(y, ht, dinv, b, W)


def _final(y, ht, dinv, b):
    _, NP, D = y.shape
    B = NP // 8

    def body(y_ref, h_ref, di_ref, b_ref, o_ref):
        s = y_ref[0] + y_ref[1] + h_ref[...]
        o_ref[...] = di_ref[...] * s + b_ref[...]

    return pl.pallas_call(
        body,
        grid=(8,),
        in_specs=[pl.BlockSpec((2, B, D), lambda i: (0, i, 0)),
                  pl.BlockSpec((B, D), lambda i: (i, 0)),
                  pl.BlockSpec((B, 1), lambda i: (i, 0)),
                  pl.BlockSpec((1, D), lambda i: (0, 0))],
        out_specs=pl.BlockSpec((B, D), lambda i: (i, 0)),
        out_shape=jax.ShapeDtypeStruct((NP, D), jnp.float32),
    )(y, ht, dinv, b)


def kernel(x, edge_index, W1, b1, W2, b2):
    N, D = x.shape
    E = edge_index.shape[1]
    NP = -(-(N + PAD_ROWS) // 256) * 256
    EPT = -(-E // (NW * CHUNK)) * CHUNK
    EPAD = EPT * NW
    padn = EPAD - E

    src = edge_index[0].astype(jnp.int32)
    dst = edge_index[1].astype(jnp.int32)
    pidx = jnp.arange(padn, dtype=jnp.int32)
    srcp = jnp.concatenate([src, (pidx * 37) % N])
    dstp = jnp.concatenate([dst, N + (pidx % PAD_ROWS)])

    xp = jnp.pad(x, ((0, NP - N), (0, 0)))
    zeros2 = jnp.zeros((NP, D), jnp.float32)

    deg_k = _make_deg_kernel(NP, EPT)
    scat_k = _make_scatter_kernel(NP, EPT, D)

    degp = deg_k(dstp).reshape(NC, NP)
    g1 = _matmul(xp, W1)
    dinv, h1t = _prep(degp.reshape(NC, NP, 1), g1)
    y1 = scat_k(h1t, srcp, dstp, zeros2)
    h2t = _mid(y1, h1t, dinv, b1.reshape(1, D), W2)
    y2 = scat_k(h2t, srcp, dstp, zeros2)
    out = _final(y2, h2t, dinv, b2.reshape(1, D))
    return out[:N]


# speedup vs baseline: 26.9239x; 1.6852x over previous
import functools

import jax
import jax.numpy as jnp
from jax import lax
from jax.experimental import pallas as pl
from jax.experimental.pallas import tpu as pltpu
from jax.experimental.pallas import tpu_sc as plsc

NC = 2
NS = 16
NW = NC * NS
LANES = 16
CHUNK = 128
PAD_ROWS = 16


def _mesh():
    return plsc.VectorSubcoreMesh(core_axis_name="c", subcore_axis_name="s")


def _make_deg_kernel(NP, EPT):
    nz = NP // NS

    @functools.partial(
        pl.kernel,
        out_type=jax.ShapeDtypeStruct((NC * NP,), jnp.float32),
        mesh=_mesh(),
        scratch_types=[
            pltpu.VMEM((2, CHUNK), jnp.int32),
            pltpu.VMEM((CHUNK,), jnp.float32),
            pltpu.VMEM((NP // NS,), jnp.float32),
            pltpu.VMEM_SHARED((NP,), jnp.float32),
            pltpu.SemaphoreType.DMA,
            pltpu.SemaphoreType.DMA,
        ],
    )
    def deg_kernel(dst_hbm, out_hbm, idx_v, ones_v, row_v, acc_sh, sem_a, sem_b):
        cid = lax.axis_index("c")
        sid = lax.axis_index("s")
        wid = sid * NC + cid
        NCH = EPT // CHUNK
        idx = (idx_v.at[0], idx_v.at[1])
        sems = (sem_a, sem_b)

        @pl.loop(0, CHUNK // LANES)
        def _(i):
            ones_v[pl.ds(i * LANES, LANES)] = jnp.full((LANES,), 1.0, jnp.float32)

        @pl.loop(0, nz // LANES)
        def _(i):
            row_v[pl.ds(i * LANES, LANES)] = jnp.zeros((LANES,), jnp.float32)

        pltpu.sync_copy(row_v, acc_sh.at[pl.ds(sid * nz, nz)])
        plsc.subcore_barrier()

        base = wid * EPT
        for b in range(2):
            pltpu.sync_copy(dst_hbm.at[pl.ds(base + b * CHUNK, CHUNK)], idx[b])
            pltpu.async_copy(ones_v, acc_sh.at[idx[b]], sems[b], add=True)

        @pl.loop(0, (NCH - 2) // 2)
        def _(i):
            c = i * 2
            for b in range(2):
                cc = c + b
                pltpu.make_async_copy(ones_v, acc_sh.at[idx[b]], sems[b]).wait()
                pltpu.sync_copy(dst_hbm.at[pl.ds(base + (cc + 2) * CHUNK, CHUNK)],
                                idx[b])
                pltpu.async_copy(ones_v, acc_sh.at[idx[b]], sems[b], add=True)

        for b in range(2):
            pltpu.make_async_copy(ones_v, acc_sh.at[idx[b]], sems[b]).wait()

        plsc.subcore_barrier()
        pltpu.sync_copy(acc_sh.at[pl.ds(sid * nz, nz)], row_v)
        pltpu.sync_copy(row_v, out_hbm.at[pl.ds(cid * NP + sid * nz, nz)])

    return deg_kernel


def _make_scatter_kernel(NP, EPT, D):
    nz = NP // NS

    NCH = EPT // CHUNK

    @functools.partial(
        pl.kernel,
        out_type=jax.ShapeDtypeStruct((NC, NP, D), jnp.float32),
        mesh=_mesh(),
        scratch_types=[
            pltpu.VMEM((2, CHUNK), jnp.int32),
            pltpu.VMEM((2, CHUNK), jnp.int32),
            pltpu.VMEM((CHUNK, D), jnp.float32),
            pltpu.VMEM((CHUNK, D), jnp.float32),
            pltpu.VMEM_SHARED((NP, D), jnp.float32),
            pltpu.SemaphoreType.DMA,
            pltpu.SemaphoreType.DMA,
            pltpu.SemaphoreType.DMA,
            pltpu.SemaphoreType.DMA,
        ],
    )
    def scat_kernel(h_hbm, sdx_hbm, zeros_hbm, out_hbm,
                    idx0, idx1, rows0, rows1, acc_sh, ga, gb, sa, sb):
        cid = lax.axis_index("c")
        sid = lax.axis_index("s")
        wid = sid * NC + cid
        idx = (idx0, idx1)
        rows = (rows0, rows1)
        gsem = (ga, gb)
        ssem = (sa, sb)
        base = wid * NCH

        for b in range(2):
            pltpu.sync_copy(sdx_hbm.at[base + b], idx[b])
            pltpu.async_copy(h_hbm.at[idx[b].at[0]], rows[b], gsem[b])

        pltpu.sync_copy(zeros_hbm.at[pl.ds(sid * nz, nz)],
                        acc_sh.at[pl.ds(sid * nz, nz)])
        plsc.subcore_barrier()

        @pl.loop(0, (NCH - 2) // 2)
        def _(i):
            c = i * 2
            for b in range(2):
                cc = c + b
                pltpu.make_async_copy(h_hbm.at[idx[b].at[0]], rows[b],
                                      gsem[b]).wait()
                pltpu.async_copy(rows[b], acc_sh.at[idx[b].at[1]], ssem[b],
                                 add=True)
                pltpu.make_async_copy(rows[b], acc_sh.at[idx[b].at[1]],
                                      ssem[b]).wait()
                pltpu.sync_copy(sdx_hbm.at[base + cc + 2], idx[b])
                pltpu.async_copy(h_hbm.at[idx[b].at[0]], rows[b], gsem[b])

        for b in range(2):
            pltpu.make_async_copy(h_hbm.at[idx[b].at[0]], rows[b],
                                  gsem[b]).wait()
            pltpu.sync_copy(rows[b], acc_sh.at[idx[b].at[1]], add=True)

        plsc.subcore_barrier()
        pltpu.sync_copy(acc_sh.at[pl.ds(sid * nz, nz)],
                        out_hbm.at[cid, pl.ds(sid * nz, nz)])

    return scat_kernel


def _matmul(x, W):
    NP, D = x.shape
    B = NP // 8

    def body(x_ref, w_ref, o_ref):
        o_ref[...] = jnp.dot(x_ref[...], w_ref[...],
                             preferred_element_type=jnp.float32)

    return pl.pallas_call(
        body,
        grid=(8,),
        in_specs=[pl.BlockSpec((B, D), lambda i: (i, 0)),
                  pl.BlockSpec((D, D), lambda i: (0, 0))],
        out_specs=pl.BlockSpec((B, D), lambda i: (i, 0)),
        out_shape=jax.ShapeDtypeStruct((NP, D), jnp.float32),
    )(x, W)


def _prep(degp, g1):
    _, NP, _ = degp.shape
    D = g1.shape[1]
    B = NP // 8

    def body(d_ref, g_ref, di_ref, h_ref):
        d = d_ref[0] + d_ref[1]
        di = lax.rsqrt(d + 1.0)
        di_ref[...] = di
        h_ref[...] = di * g_ref[...]

    return pl.pallas_call(
        body,
        grid=(8,),
        in_specs=[pl.BlockSpec((2, B, 1), lambda i: (0, i, 0)),
                  pl.BlockSpec((B, D), lambda i: (i, 0))],
        out_specs=[pl.BlockSpec((B, 1), lambda i: (i, 0)),
                   pl.BlockSpec((B, D), lambda i: (i, 0))],
        out_shape=[jax.ShapeDtypeStruct((NP, 1), jnp.float32),
                   jax.ShapeDtypeStruct((NP, D), jnp.float32)],
    )(degp, g1)


def _mid(y, ht, dinv, b, W):
    _, NP, D = y.shape
    B = NP // 8

    def body(y_ref, h_ref, di_ref, b_ref, w_ref, o_ref):
        s = y_ref[0] + y_ref[1] + h_ref[...]
        z = jnp.maximum(di_ref[...] * s + b_ref[...], 0.0)
        o_ref[...] = di_ref[...] * jnp.dot(z, w_ref[...],
                                           preferred_element_type=jnp.float32)

    return pl.pallas_call(
        body,
        grid=(8,),
        in_specs=[pl.BlockSpec((2, B, D), lambda i: (0, i, 0)),
                  pl.BlockSpec((B, D), lambda i: (i, 0)),
                  pl.BlockSpec((B, 1), lambda i: (i, 0)),
                  pl.BlockSpec((1, D), lambda i: (0, 0)),
                  pl.BlockSpec((D, D), lambda i: (0, 0))],
        out_specs=pl.BlockSpec((B, D), lambda i: (i, 0)),
        out_shape=jax.ShapeDtypeStruct((NP, D), jnp.float32),
    )(y, ht, dinv, b, W)


def _final(y, ht, dinv, b):
    _, NP, D = y.shape
    B = NP // 8

    def body(y_ref, h_ref, di_ref, b_ref, o_ref):
        s = y_ref[0] + y_ref[1] + h_ref[...]
        o_ref[...] = di_ref[...] * s + b_ref[...]

    return pl.pallas_call(
        body,
        grid=(8,),
        in_specs=[pl.BlockSpec((2, B, D), lambda i: (0, i, 0)),
                  pl.BlockSpec((B, D), lambda i: (i, 0)),
                  pl.BlockSpec((B, 1), lambda i: (i, 0)),
                  pl.BlockSpec((1, D), lambda i: (0, 0))],
        out_specs=pl.BlockSpec((B, D), lambda i: (i, 0)),
        out_shape=jax.ShapeDtypeStruct((NP, D), jnp.float32),
    )(y, ht, dinv, b)


def kernel(x, edge_index, W1, b1, W2, b2):
    N, D = x.shape
    E = edge_index.shape[1]
    NP = -(-(N + PAD_ROWS) // 256) * 256
    EPT = -(-E // (NW * 2 * CHUNK)) * (2 * CHUNK)
    EPAD = EPT * NW
    padn = EPAD - E

    src = edge_index[0].astype(jnp.int32)
    dst = edge_index[1].astype(jnp.int32)
    pidx = jnp.arange(padn, dtype=jnp.int32)
    srcp = jnp.concatenate([src, (pidx * 37) % N])
    dstp = jnp.concatenate([dst, N + (pidx % PAD_ROWS)])
    sdx = jnp.stack([srcp.reshape(-1, CHUNK), dstp.reshape(-1, CHUNK)], axis=1)

    xp = jnp.pad(x, ((0, NP - N), (0, 0)))
    zeros2 = jnp.zeros((NP, D), jnp.float32)

    deg_k = _make_deg_kernel(NP, EPT)
    scat_k = _make_scatter_kernel(NP, EPT, D)

    degp = deg_k(dstp).reshape(NC, NP)
    g1 = _matmul(xp, W1)
    dinv, h1t = _prep(degp.reshape(NC, NP, 1), g1)
    y1 = scat_k(h1t, sdx, zeros2)
    h2t = _mid(y1, h1t, dinv, b1.reshape(1, D), W2)
    y2 = scat_k(h2t, sdx, zeros2)
    out = _final(y2, h2t, dinv, b2.reshape(1, D))
    return out[:N]


# speedup vs baseline: 28.2352x; 1.0487x over previous
import functools

import jax
import jax.numpy as jnp
from jax import lax
from jax.experimental import pallas as pl
from jax.experimental.pallas import tpu as pltpu
from jax.experimental.pallas import tpu_sc as plsc

NC = 2
NS = 16
NW = NC * NS
LANES = 16
CHUNK = 128
SCH = 64
RING = 4
PAD_ROWS = 16


def _mesh():
    return plsc.VectorSubcoreMesh(core_axis_name="c", subcore_axis_name="s")


def _make_deg_kernel(NP, EPT):
    nz = NP // NS

    @functools.partial(
        pl.kernel,
        out_type=jax.ShapeDtypeStruct((NC * NP,), jnp.float32),
        mesh=_mesh(),
        scratch_types=[
            pltpu.VMEM((4, CHUNK), jnp.int32),
            pltpu.VMEM((CHUNK,), jnp.float32),
            pltpu.VMEM((NP // NS,), jnp.float32),
            pltpu.VMEM_SHARED((NP,), jnp.float32),
            pltpu.SemaphoreType.DMA,
            pltpu.SemaphoreType.DMA,
            pltpu.SemaphoreType.DMA,
            pltpu.SemaphoreType.DMA,
        ],
    )
    def deg_kernel(dstc_hbm, out_hbm, idx_v, ones_v, row_v, acc_sh,
                   s0, s1, s2, s3):
        cid = lax.axis_index("c")
        sid = lax.axis_index("s")
        wid = sid * NC + cid
        NCH = EPT // CHUNK
        dsti = tuple(idx_v.at[b] for b in range(4))
        sems = (s0, s1, s2, s3)
        base = wid * NCH

        @pl.loop(0, CHUNK // LANES)
        def _(i):
            ones_v[pl.ds(i * LANES, LANES)] = jnp.full((LANES,), 1.0, jnp.float32)

        @pl.loop(0, nz // LANES)
        def _(i):
            row_v[pl.ds(i * LANES, LANES)] = jnp.zeros((LANES,), jnp.float32)

        pltpu.sync_copy(row_v, acc_sh.at[pl.ds(sid * nz, nz)])
        plsc.subcore_barrier()

        for b in range(4):
            pltpu.sync_copy(dstc_hbm.at[base + b], dsti[b])
            pltpu.async_copy(ones_v, acc_sh.at[dsti[b]], sems[b], add=True)

        @pl.loop(0, (NCH - 4) // 4)
        def _(i):
            c = i * 4
            for b in range(4):
                cc = c + b
                pltpu.make_async_copy(ones_v, acc_sh.at[dsti[b]],
                                      sems[b]).wait()
                pltpu.sync_copy(dstc_hbm.at[base + cc + 4], dsti[b])
                pltpu.async_copy(ones_v, acc_sh.at[dsti[b]], sems[b],
                                 add=True)

        for b in range(4):
            pltpu.make_async_copy(ones_v, acc_sh.at[dsti[b]], sems[b]).wait()

        plsc.subcore_barrier()
        pltpu.sync_copy(acc_sh.at[pl.ds(sid * nz, nz)], row_v)
        pltpu.sync_copy(row_v, out_hbm.at[pl.ds(cid * NP + sid * nz, nz)])

    return deg_kernel


def _make_scatter_kernel(NP, EPT, D):
    nz = NP // NS

    NCH = EPT // SCH

    @functools.partial(
        pl.kernel,
        out_type=jax.ShapeDtypeStruct((NC, NP, D), jnp.float32),
        mesh=_mesh(),
        scratch_types=[
            pltpu.VMEM((2 * RING, SCH), jnp.int32),
            pltpu.VMEM((SCH, D), jnp.float32),
            pltpu.VMEM((SCH, D), jnp.float32),
            pltpu.VMEM((SCH, D), jnp.float32),
            pltpu.VMEM((SCH, D), jnp.float32),
            pltpu.VMEM_SHARED((NP, D), jnp.float32),
        ] + [pltpu.SemaphoreType.DMA] * 8,
    )
    def scat_kernel(h_hbm, sdx_hbm, zeros_hbm, out_hbm,
                    idx_v, rows0, rows1, rows2, rows3, acc_sh,
                    g0, g1, g2, g3, s0, s1, s2, s3):
        cid = lax.axis_index("c")
        sid = lax.axis_index("s")
        wid = sid * NC + cid
        ld = tuple(idx_v.at[pl.ds(2 * b, 2)] for b in range(4))
        srci = tuple(idx_v.at[2 * b] for b in range(4))
        dsti = tuple(idx_v.at[2 * b + 1] for b in range(4))
        rows = (rows0, rows1, rows2, rows3)
        gsem = (g0, g1, g2, g3)
        ssem = (s0, s1, s2, s3)
        base = wid * NCH

        def load_gather(cc, b):
            pltpu.sync_copy(sdx_hbm.at[base + cc], ld[b])
            pltpu.async_copy(h_hbm.at[srci[b]], rows[b], gsem[b])

        def wait_gather_scatter(cc, b):
            pltpu.make_async_copy(h_hbm.at[srci[b]], rows[b], gsem[b]).wait()
            pltpu.async_copy(rows[b], acc_sh.at[dsti[b]], ssem[b], add=True)

        def wait_scatter(b):
            pltpu.make_async_copy(rows[b], acc_sh.at[dsti[b]], ssem[b]).wait()

        for b in range(2):
            load_gather(b, b)

        pltpu.sync_copy(zeros_hbm.at[pl.ds(sid * nz, nz)],
                        acc_sh.at[pl.ds(sid * nz, nz)])
        plsc.subcore_barrier()

        for cc in range(2):
            load_gather(cc + 2, cc + 2)
            wait_gather_scatter(cc, cc)

        @pl.loop(0, (NCH - 4) // 4)
        def _(i):
            c = 2 + i * 4
            for db in range(4):
                cc = c + db
                b = (2 + db) % 4
                bL = db
                wait_scatter(bL)
                load_gather(cc + 2, bL)
                wait_gather_scatter(cc, b)

        for db in range(2):
            wait_gather_scatter(NCH - 2 + db, 2 + db)
        for b in range(4):
            wait_scatter(b)

        plsc.subcore_barrier()
        pltpu.sync_copy(acc_sh.at[pl.ds(sid * nz, nz)],
                        out_hbm.at[cid, pl.ds(sid * nz, nz)])

    return scat_kernel


def _matmul(x, W):
    NP, D = x.shape
    B = NP // 8

    def body(x_ref, w_ref, o_ref):
        o_ref[...] = jnp.dot(x_ref[...], w_ref[...],
                             preferred_element_type=jnp.float32)

    return pl.pallas_call(
        body,
        grid=(8,),
        in_specs=[pl.BlockSpec((B, D), lambda i: (i, 0)),
                  pl.BlockSpec((D, D), lambda i: (0, 0))],
        out_specs=pl.BlockSpec((B, D), lambda i: (i, 0)),
        out_shape=jax.ShapeDtypeStruct((NP, D), jnp.float32),
    )(x, W)


def _prep(degp, g1):
    _, NP, _ = degp.shape
    D = g1.shape[1]
    B = NP // 8

    def body(d_ref, g_ref, di_ref, h_ref):
        d = d_ref[0] + d_ref[1]
        di = lax.rsqrt(d + 1.0)
        di_ref[...] = di
        h_ref[...] = di * g_ref[...]

    return pl.pallas_call(
        body,
        grid=(8,),
        in_specs=[pl.BlockSpec((2, B, 1), lambda i: (0, i, 0)),
                  pl.BlockSpec((B, D), lambda i: (i, 0))],
        out_specs=[pl.BlockSpec((B, 1), lambda i: (i, 0)),
                   pl.BlockSpec((B, D), lambda i: (i, 0))],
        out_shape=[jax.ShapeDtypeStruct((NP, 1), jnp.float32),
                   jax.ShapeDtypeStruct((NP, D), jnp.float32)],
    )(degp, g1)


def _mid(y, ht, dinv, b, W):
    _, NP, D = y.shape
    B = NP // 8

    def body(y_ref, h_ref, di_ref, b_ref, w_ref, o_ref):
        s = y_ref[0] + y_ref[1] + h_ref[...]
        z = jnp.maximum(di_ref[...] * s + b_ref[...], 0.0)
        o_ref[...] = di_ref[...] * jnp.dot(z, w_ref[...],
                                           preferred_element_type=jnp.float32)

    return pl.pallas_call(
        body,
        grid=(8,),
        in_specs=[pl.BlockSpec((2, B, D), lambda i: (0, i, 0)),
                  pl.BlockSpec((B, D), lambda i: (i, 0)),
                  pl.BlockSpec((B, 1), lambda i: (i, 0)),
                  pl.BlockSpec((1, D), lambda i: (0, 0)),
                  pl.BlockSpec((D, D), lambda i: (0, 0))],
        out_specs=pl.BlockSpec((B, D), lambda i: (i, 0)),
        out_shape=jax.ShapeDtypeStruct((NP, D), jnp.float32),
    )(y, ht, dinv, b, W)


def _final(y, ht, dinv, b):
    _, NP, D = y.shape
    B = NP // 8

    def body(y_ref, h_ref, di_ref, b_ref, o_ref):
        s = y_ref[0] + y_ref[1] + h_ref[...]
        o_ref[...] = di_ref[...] * s + b_ref[...]

    return pl.pallas_call(
        body,
        grid=(8,),
        in_specs=[pl.BlockSpec((2, B, D), lambda i: (0, i, 0)),
                  pl.BlockSpec((B, D), lambda i: (i, 0)),
                  pl.BlockSpec((B, 1), lambda i: (i, 0)),
                  pl.BlockSpec((1, D), lambda i: (0, 0))],
        out_specs=pl.BlockSpec((B, D), lambda i: (i, 0)),
        out_shape=jax.ShapeDtypeStruct((NP, D), jnp.float32),
    )(y, ht, dinv, b)


def kernel(x, edge_index, W1, b1, W2, b2):
    N, D = x.shape
    E = edge_index.shape[1]
    NP = -(-(N + PAD_ROWS) // 256) * 256
    EPT = -(-E // (NW * 4 * CHUNK)) * (4 * CHUNK)
    EPAD = EPT * NW
    padn = EPAD - E

    src = edge_index[0].astype(jnp.int32)
    dst = edge_index[1].astype(jnp.int32)
    pidx = jnp.arange(padn, dtype=jnp.int32)
    srcp = jnp.concatenate([src, (pidx * 37) % N])
    dstp = jnp.concatenate([dst, N + (pidx % PAD_ROWS)])
    sdx = jnp.stack([srcp.reshape(-1, SCH), dstp.reshape(-1, SCH)], axis=1)
    dstc = dstp.reshape(-1, CHUNK)

    xp = jnp.pad(x, ((0, NP - N), (0, 0)))
    zeros2 = jnp.zeros((NP, D), jnp.float32)

    deg_k = _make_deg_kernel(NP, EPT)
    scat_k = _make_scatter_kernel(NP, EPT, D)

    degp = deg_k(dstc).reshape(NC, NP)
    g1 = _matmul(xp, W1)
    dinv, h1t = _prep(degp.reshape(NC, NP, 1), g1)
    y1 = scat_k(h1t, sdx, zeros2)
    h2t = _mid(y1, h1t, dinv, b1.reshape(1, D), W2)
    y2 = scat_k(h2t, sdx, zeros2)
    out = _final(y2, h2t, dinv, b2.reshape(1, D))
    return out[:N]


# speedup vs baseline: 29.7993x; 1.0554x over previous
import functools

import jax
import jax.numpy as jnp
from jax import lax
from jax.experimental import pallas as pl
from jax.experimental.pallas import tpu as pltpu
from jax.experimental.pallas import tpu_sc as plsc

NC = 2
NS = 16
NW = NC * NS
LANES = 16
CHUNK = 128
SCH = 64
RING = 4
PAD_ROWS = 16


def _mesh():
    return plsc.VectorSubcoreMesh(core_axis_name="c", subcore_axis_name="s")


def _make_deg_kernel(NP, EPT0):
    nz = NP // NS
    NCH = EPT0 // CHUNK
    NCHR = (NCH // 4) * 4
    TAIL = EPT0 - NCH * CHUNK

    @functools.partial(
        pl.kernel,
        out_type=jax.ShapeDtypeStruct((NC * NP,), jnp.float32),
        mesh=_mesh(),
        scratch_types=[
            pltpu.VMEM((4, CHUNK), jnp.int32),
            pltpu.VMEM((max(TAIL, 8),), jnp.int32),
            pltpu.VMEM((CHUNK,), jnp.float32),
            pltpu.VMEM((NP // NS,), jnp.float32),
            pltpu.VMEM_SHARED((NP,), jnp.float32),
            pltpu.SemaphoreType.DMA,
            pltpu.SemaphoreType.DMA,
            pltpu.SemaphoreType.DMA,
            pltpu.SemaphoreType.DMA,
        ],
    )
    def deg_kernel(dst_hbm, out_hbm, idx_v, tidx_v, ones_v, row_v, acc_sh,
                   s0, s1, s2, s3):
        cid = lax.axis_index("c")
        sid = lax.axis_index("s")
        wid = sid * NC + cid
        dsti = tuple(idx_v.at[b] for b in range(4))
        sems = (s0, s1, s2, s3)
        base = wid * EPT0

        @pl.loop(0, CHUNK // LANES)
        def _(i):
            ones_v[pl.ds(i * LANES, LANES)] = jnp.full((LANES,), 1.0, jnp.float32)

        @pl.loop(0, nz // LANES)
        def _(i):
            row_v[pl.ds(i * LANES, LANES)] = jnp.zeros((LANES,), jnp.float32)

        pltpu.sync_copy(row_v, acc_sh.at[pl.ds(sid * nz, nz)])
        plsc.subcore_barrier()

        for b in range(4):
            pltpu.sync_copy(dst_hbm.at[pl.ds(base + b * CHUNK, CHUNK)], dsti[b])
            pltpu.async_copy(ones_v, acc_sh.at[dsti[b]], sems[b], add=True)

        @pl.loop(0, (NCHR - 4) // 4)
        def _(i):
            c = i * 4
            for b in range(4):
                cc = c + b
                pltpu.make_async_copy(ones_v, acc_sh.at[dsti[b]],
                                      sems[b]).wait()
                pltpu.sync_copy(
                    dst_hbm.at[pl.ds(base + (cc + 4) * CHUNK, CHUNK)], dsti[b])
                pltpu.async_copy(ones_v, acc_sh.at[dsti[b]], sems[b],
                                 add=True)

        for b in range(4):
            pltpu.make_async_copy(ones_v, acc_sh.at[dsti[b]], sems[b]).wait()

        for cc in range(NCHR, NCH):
            pltpu.sync_copy(dst_hbm.at[pl.ds(base + cc * CHUNK, CHUNK)],
                            dsti[0])
            pltpu.sync_copy(ones_v, acc_sh.at[dsti[0]], add=True)
        if TAIL:
            pltpu.sync_copy(dst_hbm.at[pl.ds(base + NCH * CHUNK, TAIL)], tidx_v)
            pltpu.sync_copy(ones_v.at[pl.ds(0, TAIL)], acc_sh.at[tidx_v],
                            add=True)

        plsc.subcore_barrier()
        pltpu.sync_copy(acc_sh.at[pl.ds(sid * nz, nz)], row_v)
        pltpu.sync_copy(row_v, out_hbm.at[pl.ds(cid * NP + sid * nz, nz)])

    return deg_kernel


def _make_scatter_kernel(NP, EPT, D):
    nz = NP // NS
    NCH = EPT // SCH

    @functools.partial(
        pl.kernel,
        out_type=jax.ShapeDtypeStruct((NC, NP, D), jnp.float32),
        mesh=_mesh(),
        scratch_types=[
            pltpu.VMEM((2 * RING, SCH), jnp.int32),
            pltpu.VMEM((SCH, D), jnp.float32),
            pltpu.VMEM((SCH, D), jnp.float32),
            pltpu.VMEM((SCH, D), jnp.float32),
            pltpu.VMEM((SCH, D), jnp.float32),
            pltpu.VMEM_SHARED((NP, D), jnp.float32),
        ] + [pltpu.SemaphoreType.DMA] * 8,
    )
    def scat_kernel(h_hbm, sdx_hbm, zeros_hbm, out_hbm,
                    idx_v, rows0, rows1, rows2, rows3, acc_sh,
                    g0, g1, g2, g3, s0, s1, s2, s3):
        cid = lax.axis_index("c")
        sid = lax.axis_index("s")
        wid = sid * NC + cid
        ld = tuple(idx_v.at[pl.ds(2 * b, 2)] for b in range(4))
        srci = tuple(idx_v.at[2 * b] for b in range(4))
        dsti = tuple(idx_v.at[2 * b + 1] for b in range(4))
        rows = (rows0, rows1, rows2, rows3)
        gsem = (g0, g1, g2, g3)
        ssem = (s0, s1, s2, s3)
        base = wid * NCH

        def load_gather(cc, b):
            pltpu.sync_copy(sdx_hbm.at[base + cc], ld[b])
            pltpu.async_copy(h_hbm.at[srci[b]], rows[b], gsem[b])

        def wait_gather_scatter(cc, b):
            pltpu.make_async_copy(h_hbm.at[srci[b]], rows[b], gsem[b]).wait()
            pltpu.async_copy(rows[b], acc_sh.at[dsti[b]], ssem[b], add=True)

        def wait_scatter(b):
            pltpu.make_async_copy(rows[b], acc_sh.at[dsti[b]], ssem[b]).wait()

        for b in range(2):
            load_gather(b, b)

        pltpu.sync_copy(zeros_hbm.at[pl.ds(sid * nz, nz)],
                        acc_sh.at[pl.ds(sid * nz, nz)])
        plsc.subcore_barrier()

        for cc in range(2):
            load_gather(cc + 2, cc + 2)
            wait_gather_scatter(cc, cc)

        @pl.loop(0, (NCH - 4) // 4)
        def _(i):
            c = 2 + i * 4
            for db in range(4):
                cc = c + db
                b = (2 + db) % 4
                bL = db
                wait_scatter(bL)
                load_gather(cc + 2, bL)
                wait_gather_scatter(cc, b)

        for db in range(2):
            wait_gather_scatter(NCH - 2 + db, 2 + db)
        for b in range(4):
            wait_scatter(b)

        plsc.subcore_barrier()
        pltpu.sync_copy(acc_sh.at[pl.ds(sid * nz, nz)],
                        out_hbm.at[cid, pl.ds(sid * nz, nz)])

    return scat_kernel


def _matmul(x, W):
    NP, D = x.shape
    B = NP // 8

    def body(x_ref, w_ref, o_ref):
        o_ref[...] = jnp.dot(x_ref[...], w_ref[...],
                             preferred_element_type=jnp.float32)

    return pl.pallas_call(
        body,
        grid=(8,),
        in_specs=[pl.BlockSpec((B, D), lambda i: (i, 0)),
                  pl.BlockSpec((D, D), lambda i: (0, 0))],
        out_specs=pl.BlockSpec((B, D), lambda i: (i, 0)),
        out_shape=jax.ShapeDtypeStruct((NP, D), jnp.float32),
    )(x, W)


def _prep(degf, g1):
    _, NPL, _ = degf.shape
    NP, D = g1.shape
    B = NP // 10
    BL = NPL // 10

    def body(d_ref, g_ref, di_ref, h_ref):
        d = d_ref[0] + d_ref[1]
        di = lax.rsqrt(d + 1.0)
        dit = di.T
        dib = jnp.concatenate(
            [jnp.broadcast_to(dit[:, a:a + 1], (128, D)) for a in range(BL)],
            axis=0)
        di_ref[...] = dib
        h_ref[...] = dib * g_ref[...]

    return pl.pallas_call(
        body,
        grid=(10,),
        in_specs=[pl.BlockSpec((2, BL, 128), lambda i: (0, i, 0)),
                  pl.BlockSpec((B, D), lambda i: (i, 0))],
        out_specs=[pl.BlockSpec((B, D), lambda i: (i, 0)),
                   pl.BlockSpec((B, D), lambda i: (i, 0))],
        out_shape=[jax.ShapeDtypeStruct((NP, D), jnp.float32),
                   jax.ShapeDtypeStruct((NP, D), jnp.float32)],
    )(degf, g1)


def _mid(y, ht, dinvb, b, W):
    _, NP, D = y.shape
    B = NP // 8

    def body(y_ref, h_ref, di_ref, b_ref, w_ref, o_ref):
        s = y_ref[0] + y_ref[1] + h_ref[...]
        z = jnp.maximum(di_ref[...] * s + b_ref[...], 0.0)
        o_ref[...] = di_ref[...] * jnp.dot(z, w_ref[...],
                                           preferred_element_type=jnp.float32)

    return pl.pallas_call(
        body,
        grid=(8,),
        in_specs=[pl.BlockSpec((2, B, D), lambda i: (0, i, 0)),
                  pl.BlockSpec((B, D), lambda i: (i, 0)),
                  pl.BlockSpec((B, D), lambda i: (i, 0)),
                  pl.BlockSpec((1, D), lambda i: (0, 0)),
                  pl.BlockSpec((D, D), lambda i: (0, 0))],
        out_specs=pl.BlockSpec((B, D), lambda i: (i, 0)),
        out_shape=jax.ShapeDtypeStruct((NP, D), jnp.float32),
    )(y, ht, dinvb, b, W)


def _final(y, ht, dinvb, b, N):
    _, NP, D = y.shape
    B = N // 10

    def body(y_ref, h_ref, di_ref, b_ref, o_ref):
        s = y_ref[0] + y_ref[1] + h_ref[...]
        o_ref[...] = di_ref[...] * s + b_ref[...]

    return pl.pallas_call(
        body,
        grid=(10,),
        in_specs=[pl.BlockSpec((2, B, D), lambda i: (0, i, 0)),
                  pl.BlockSpec((B, D), lambda i: (i, 0)),
                  pl.BlockSpec((B, D), lambda i: (i, 0)),
                  pl.BlockSpec((1, D), lambda i: (0, 0))],
        out_specs=pl.BlockSpec((B, D), lambda i: (i, 0)),
        out_shape=jax.ShapeDtypeStruct((N, D), jnp.float32),
    )(y, ht, dinvb, b)


def kernel(x, edge_index, W1, b1, W2, b2):
    N, D = x.shape
    E = edge_index.shape[1]
    NP = -(-(N + PAD_ROWS) // 1024) * 1024
    EPT0 = E // NW
    EPT = -(-EPT0 // (4 * SCH)) * (4 * SCH)
    padn = EPT * NW - E

    ei = edge_index.astype(jnp.int32)
    src = ei[0]
    dst = ei[1]
    pidx = jnp.arange(padn, dtype=jnp.int32)
    srcp = jnp.concatenate([src, (pidx * 37) % N])
    dstp = jnp.concatenate([dst, N + (pidx % PAD_ROWS)])
    sdx = jnp.stack([srcp.reshape(-1, SCH), dstp.reshape(-1, SCH)], axis=1)

    xp = jnp.pad(x, ((0, NP - N), (0, 0)))
    zeros2 = jnp.zeros((NP, D), jnp.float32)

    deg_k = _make_deg_kernel(NP, EPT0)
    scat_k = _make_scatter_kernel(NP, EPT, D)

    degf = deg_k(dst).reshape(NC, NP // 128, 128)
    g1 = _matmul(xp, W1)
    dinvb, h1t = _prep(degf, g1)
    y1 = scat_k(h1t, sdx, zeros2)
    h2t = _mid(y1, h1t, dinvb, b1.reshape(1, D), W2)
    y2 = scat_k(h2t, sdx, zeros2)
    return _final(y2, h2t, dinvb, b2.reshape(1, D), N)
